# baseline (speedup 1.0000x reference)
import jax
import jax.numpy as jnp
from jax import lax
from jax.experimental import pallas as pl
from jax.experimental.pallas import tpu as pltpu

_HI = lax.Precision.HIGHEST

_B = 200
_V = 48
_H = 64
_NUM_LAYERS = 2
_NUM_SW = 10
_M_EDGES = (_V - 1) + _NUM_SW
_NB = 4
_G = _NB * _V
_WU = 64
_WS = 16
_WB = 48
_EU = _NB * _WU
_ES = _NB * _WS
_EB = _NB * _WB


def _onehots(tU, tL, cmats, n_slots, window, f32, transposed=False):
    G = tU.shape[0]
    c = jnp.dot(tU, cmats['right_incl'], preferred_element_type=f32)
    rtot_col = jnp.sum(tU, axis=1, keepdims=True)
    roff_col = jnp.dot(cmats['tril_bd'], rtot_col, preferred_element_type=f32)
    rtot_row = jnp.sum(tL, axis=0, keepdims=True)
    roff_row = jnp.dot(rtot_row, cmats['triu_bd'], preferred_element_type=f32)

    k_col = (lax.broadcasted_iota(jnp.int32, (n_slots, 1), 0)
             % window).astype(f32) + 1.0
    blk = ((lax.broadcasted_iota(jnp.int32, (n_slots, G), 0) // window) ==
           (lax.broadcasted_iota(jnp.int32, (n_slots, G), 1) // _V))
    row_oh = ((k_col > roff_row) & (k_col <= roff_row + rtot_row)
              & blk).astype(f32)
    rank = k_col - jnp.dot(row_oh, roff_col, preferred_element_type=f32)
    csel = jnp.dot(row_oh, c, preferred_element_type=f32)
    tsel = jnp.dot(row_oh, tU, preferred_element_type=f32)
    col_oh = jnp.where(jnp.abs(csel - rank) < 0.5, tsel, 0.0)
    iota_col = cmats['iota_col']
    i_col = jnp.dot(row_oh, iota_col, preferred_element_type=f32)
    j_col = jnp.dot(col_oh, iota_col, preferred_element_type=f32)
    fp_col = i_col * G + j_col
    if not transposed:
        return row_oh, col_oh, fp_col

    k_row = (lax.broadcasted_iota(jnp.int32, (1, n_slots), 1)
             % window).astype(f32) + 1.0
    blkT = ((lax.broadcasted_iota(jnp.int32, (G, n_slots), 0) // _V) ==
            (lax.broadcasted_iota(jnp.int32, (G, n_slots), 1) // window))
    row_ohT = ((k_row > roff_col) & (k_row <= roff_col + rtot_col)
               & blkT).astype(f32)
    cT = jnp.dot(cmats['left_incl'], tL, preferred_element_type=f32)
    rankT = k_row - jnp.dot(roff_row, row_ohT, preferred_element_type=f32)
    cselT = jnp.dot(cT, row_ohT, preferred_element_type=f32)
    tselT = jnp.dot(tL, row_ohT, preferred_element_type=f32)
    col_ohT = jnp.where(jnp.abs(cselT - rankT) < 0.5, tselT, 0.0)
    iota_row = cmats['iota_row']
    i_row = jnp.dot(iota_row, row_ohT, preferred_element_type=f32)
    j_row = jnp.dot(iota_row, col_ohT, preferred_element_type=f32)
    fp_row = i_row * G + j_row
    return row_oh, col_oh, fp_col, row_ohT, col_ohT, fp_row


def _fwd_kernel(x_ref, a_ref, s_ref, emb_ref, wuv_ref, wge_ref, we12_ref,
                sW1_ref, sb1_ref, sW2_ref, sb2_ref,
                cW1_ref, cb1_ref, cW2_ref, cb2_ref, out_ref):
    G, V, H = _G, _V, _H
    f32 = jnp.float32

    Ab = a_ref[0]
    Sb = s_ref[0]
    x_all = x_ref[...].reshape(G, H)

    mask = ((Ab + Sb) > 0).astype(f32)
    inv_deg = 1.0 / (jnp.sum(mask, axis=1, keepdims=True) + 1e-6)

    e0 = emb_ref[0:1, :]
    e1 = emb_ref[1:2, :]

    gi = lax.broadcasted_iota(jnp.int32, (G, G), 0)
    gj = lax.broadcasted_iota(jnp.int32, (G, G), 1)
    same = (gi // V) == (gj // V)
    triu_bd = (same & (gj > gi)).astype(f32)
    tril_bd = (same & (gi > gj)).astype(f32)
    cmats = {
        'right_incl': (same & (gi <= gj)).astype(f32),
        'left_incl': (same & (gj <= gi)).astype(f32),
        'triu_bd': triu_bd,
        'tril_bd': tril_bd,
        'iota_col': lax.broadcasted_iota(jnp.int32, (G, 1), 0).astype(f32),
        'iota_row': lax.broadcasted_iota(jnp.int32, (1, G), 1).astype(f32),
    }

    tU = mask * triu_bd
    tL = mask * tril_bd
    rowU, colU, fpU_col, rowUT, colUT, fpU_row = _onehots(
        tU, tL, cmats, _EU, _WU, f32, transposed=True)
    row_oh = jnp.concatenate([rowU, colU], axis=0)
    col_oh = jnp.concatenate([colU, rowU], axis=0)
    row_ohT = jnp.concatenate([rowUT, colUT], axis=1)

    tS = Sb * triu_bd
    tSL = Sb * tril_bd
    rowS, colS, fpS_col, rowST, colST, fpS_row = _onehots(
        tS, tSL, cmats, _ES, _WS, f32, transposed=True)
    ohSW = (jnp.abs(fpS_col - fpU_row) < 0.5).astype(f32)
    ohSW_T = (jnp.abs(fpU_col - fpS_row) < 0.5).astype(f32)

    Sval_u = jnp.dot(ohSW_T, jnp.ones((_ES, 1), f32),
                     preferred_element_type=f32)
    Sval = jnp.concatenate([Sval_u, Sval_u], axis=0)
    s_e = e0 + Sval * (e1 - e0)

    for l in range(_NUM_LAYERS):
        uv = jnp.dot(x_all, wuv_ref[l], preferred_element_type=f32)
        Ux = uv[:, :H]
        Vx = uv[:, H:]
        ge = jnp.dot(s_e, wge_ref[l], preferred_element_type=f32)
        gates = jax.nn.sigmoid(ge[:, :H])
        sE0 = ge[:, H:]
        Vxc = jnp.dot(col_oh, Vx, preferred_element_type=f32, precision=_HI)
        contrib = gates * Vxc
        agg = jnp.dot(row_ohT, contrib, preferred_element_type=f32, precision=_HI)
        x_all = jnp.maximum(Ux + agg * inv_deg, 0.0)
        e12 = jnp.dot(x_all, we12_ref[l], preferred_element_type=f32)
        xE1e = jnp.dot(row_oh, e12[:, :H], preferred_element_type=f32, precision=_HI)
        xE2e = jnp.dot(col_oh, e12[:, H:], preferred_element_type=f32, precision=_HI)
        s_e = jnp.maximum(sE0 + xE1e + xE2e, 0.0)

    ks = lax.broadcasted_iota(jnp.int32, (_ES, G), 0)
    gs = lax.broadcasted_iota(jnp.int32, (_ES, G), 1)
    ind_sw = ((gs // V) == (ks // _WS)).astype(f32)
    kb = lax.broadcasted_iota(jnp.int32, (_EB, G), 0)
    gb = lax.broadcasted_iota(jnp.int32, (_EB, G), 1)
    ind_br = ((gb // V) == (kb // _WB)).astype(f32)
    xg_sw = jnp.dot(ind_sw, x_all, preferred_element_type=f32, precision=_HI)
    xg_br = jnp.dot(ind_br, x_all, preferred_element_type=f32, precision=_HI)

    sw = jnp.dot(ohSW, s_e[:_EU], preferred_element_type=f32)
    x1 = jnp.dot(rowS, x_all, preferred_element_type=f32)
    x2 = jnp.dot(colS, x_all, preferred_element_type=f32)
    smlp_in = jnp.concatenate([sw, x1, x2, xg_sw], axis=1)
    hs = jnp.maximum(
        jnp.dot(smlp_in, sW1_ref[...], preferred_element_type=f32)
        + sb1_ref[...], 0.0)
    s_out = jnp.dot(hs, sW2_ref[...],
                    preferred_element_type=f32) + sb2_ref[...]

    tA = Ab * triu_bd
    tAL = Ab * tril_bd
    rowA, colA, _ = _onehots(tA, tAL, cmats, _EB, _WB, f32)
    xb = jnp.dot(rowA, x_all, preferred_element_type=f32)
    xe = jnp.dot(colA, x_all, preferred_element_type=f32)
    cmlp_in = jnp.concatenate([xb, xe, xg_br], axis=1)
    hc = jnp.maximum(
        jnp.dot(cmlp_in, cW1_ref[...], preferred_element_type=f32)
        + cb1_ref[...], 0.0)
    c_out = jnp.dot(hc, cW2_ref[...],
                    preferred_element_type=f32) + cb2_ref[...]

    nsw = _NUM_SW
    nbr = _V - 1
    zeros47 = jnp.zeros((nbr, 1), f32)
    for nb in range(_NB):
        co = c_out[nb * _WB:nb * _WB + nbr]
        so = s_out[nb * _WS:nb * _WS + nsw]
        col = jnp.concatenate([
            co[:, 0:1], so[:, 1:2],
            zeros47, jax.nn.sigmoid(so[:, 0:1]),
            co[:, 1:2], so[:, 2:3],
            co[:, 2:3], so[:, 3:4],
        ], axis=0)
        out_ref[0, :, nb:nb + 1] = col


@jax.jit
def kernel(x, A, S, params):
    f32 = jnp.float32
    H = _H
    lp = params['layers']
    wuv = jnp.stack([jnp.concatenate([l['U'], l['Vm']], axis=1) for l in lp])
    wge = jnp.stack([jnp.concatenate([l['G'], l['E0']], axis=1) for l in lp])
    we12 = jnp.stack([jnp.concatenate([l['E1'], l['E2']], axis=1) for l in lp])
    emb = params['embed']
    sW1 = params['smlp_W1']
    sb1 = params['smlp_b1'].reshape(1, 4 * H)
    sW2 = jnp.zeros((4 * H, 8), f32).at[:, :4].set(params['smlp_W2'])
    sb2 = jnp.zeros((1, 8), f32).at[0, :4].set(params['smlp_b2'])
    cW1 = params['cmlp_W1']
    cb1 = params['cmlp_b1'].reshape(1, 3 * H)
    cW2 = jnp.zeros((3 * H, 8), f32).at[:, :3].set(params['cmlp_W2'])
    cb2 = jnp.zeros((1, 8), f32).at[0, :3].set(params['cmlp_b2'])

    nprog = _B // _NB
    A4 = A.reshape(nprog, _NB, _V, _V)
    S4 = S.reshape(nprog, _NB, _V, _V)
    Abig = jnp.zeros((nprog, _G, _G), f32)
    Sbig = jnp.zeros((nprog, _G, _G), f32)
    for nb in range(_NB):
        sl = slice(nb * _V, (nb + 1) * _V)
        Abig = Abig.at[:, sl, sl].set(A4[:, nb])
        Sbig = Sbig.at[:, sl, sl].set(S4[:, nb])

    grid = (nprog,)
    full = lambda shape: pl.BlockSpec(shape, lambda i: (0,) * len(shape))
    out = pl.pallas_call(
        _fwd_kernel,
        grid=grid,
        in_specs=[
            pl.BlockSpec((_NB, _V, _H), lambda i: (i, 0, 0)),
            pl.BlockSpec((1, _G, _G), lambda i: (i, 0, 0)),
            pl.BlockSpec((1, _G, _G), lambda i: (i, 0, 0)),
            full((2, H)),
            full((_NUM_LAYERS, H, 2 * H)),
            full((_NUM_LAYERS, H, 2 * H)),
            full((_NUM_LAYERS, H, 2 * H)),
            full((4 * H, 4 * H)),
            full((1, 4 * H)),
            full((4 * H, 8)),
            full((1, 8)),
            full((3 * H, 3 * H)),
            full((1, 3 * H)),
            full((3 * H, 8)),
            full((1, 8)),
        ],
        out_specs=pl.BlockSpec((1, 4 * _M_EDGES, _NB), lambda i: (i, 0, 0)),
        out_shape=jax.ShapeDtypeStruct((nprog, 4 * _M_EDGES, _NB), f32),
        compiler_params=pltpu.CompilerParams(
            dimension_semantics=("parallel",)),
    )(x, Abig, Sbig, emb, wuv, wge, we12, sW1, sb1, sW2, sb2,
      cW1, cb1, cW2, cb2)
    return out.transpose(0, 2, 1).reshape(_B, 4 * _M_EDGES)


# speedup vs baseline: 1.2840x; 1.2840x over previous
import jax
import jax.numpy as jnp
from jax import lax
from jax.experimental import pallas as pl
from jax.experimental.pallas import tpu as pltpu

_HI = lax.Precision.HIGHEST


def _dot2(oh, v, f32):
    hi = v.astype(jnp.bfloat16).astype(f32)
    lo = v - hi
    return (jnp.dot(oh, hi, preferred_element_type=f32)
            + jnp.dot(oh, lo, preferred_element_type=f32))

_B = 200
_V = 48
_H = 64
_NUM_LAYERS = 2
_NUM_SW = 10
_M_EDGES = (_V - 1) + _NUM_SW
_NB = 4
_G = _NB * _V
_WU = 64
_WS = 16
_WB = 48
_EU = _NB * _WU
_ES = _NB * _WS
_EB = _NB * _WB


def _onehots(tU, tL, cmats, n_slots, window, f32, transposed=False):
    G = tU.shape[0]
    c = jnp.dot(tU, cmats['right_incl'], preferred_element_type=f32)
    rtot_col = jnp.sum(tU, axis=1, keepdims=True)
    roff_col = jnp.dot(cmats['tril_bd'], rtot_col, preferred_element_type=f32)
    rtot_row = jnp.sum(tL, axis=0, keepdims=True)
    roff_row = jnp.dot(rtot_row, cmats['triu_bd'], preferred_element_type=f32)

    k_col = (lax.broadcasted_iota(jnp.int32, (n_slots, 1), 0)
             % window).astype(f32) + 1.0
    blk = ((lax.broadcasted_iota(jnp.int32, (n_slots, G), 0) // window) ==
           (lax.broadcasted_iota(jnp.int32, (n_slots, G), 1) // _V))
    row_oh = ((k_col > roff_row) & (k_col <= roff_row + rtot_row)
              & blk).astype(f32)
    rank = k_col - jnp.dot(row_oh, roff_col, preferred_element_type=f32)
    csel = jnp.dot(row_oh, c, preferred_element_type=f32)
    tsel = jnp.dot(row_oh, tU, preferred_element_type=f32)
    col_oh = jnp.where(jnp.abs(csel - rank) < 0.5, tsel, 0.0)
    iota_col = cmats['iota_col']
    i_col = jnp.dot(row_oh, iota_col, preferred_element_type=f32)
    j_col = jnp.dot(col_oh, iota_col, preferred_element_type=f32)
    fp_col = i_col * G + j_col
    if not transposed:
        return row_oh, col_oh, fp_col

    k_row = (lax.broadcasted_iota(jnp.int32, (1, n_slots), 1)
             % window).astype(f32) + 1.0
    blkT = ((lax.broadcasted_iota(jnp.int32, (G, n_slots), 0) // _V) ==
            (lax.broadcasted_iota(jnp.int32, (G, n_slots), 1) // window))
    row_ohT = ((k_row > roff_col) & (k_row <= roff_col + rtot_col)
               & blkT).astype(f32)
    cT = jnp.dot(cmats['left_incl'], tL, preferred_element_type=f32)
    rankT = k_row - jnp.dot(roff_row, row_ohT, preferred_element_type=f32)
    cselT = jnp.dot(cT, row_ohT, preferred_element_type=f32)
    tselT = jnp.dot(tL, row_ohT, preferred_element_type=f32)
    col_ohT = jnp.where(jnp.abs(cselT - rankT) < 0.5, tselT, 0.0)
    iota_row = cmats['iota_row']
    i_row = jnp.dot(iota_row, row_ohT, preferred_element_type=f32)
    j_row = jnp.dot(iota_row, col_ohT, preferred_element_type=f32)
    fp_row = i_row * G + j_row
    return row_oh, col_oh, fp_col, row_ohT, col_ohT, fp_row


def _fwd_kernel(x_ref, a_ref, s_ref, emb_ref, wuv_ref, wge_ref, we12_ref,
                sW1_ref, sb1_ref, sW2_ref, sb2_ref,
                cW1_ref, cb1_ref, cW2_ref, cb2_ref, out_ref):
    G, V, H = _G, _V, _H
    f32 = jnp.float32

    Ab = a_ref[0]
    Sb = s_ref[0]
    x_all = x_ref[...].reshape(G, H)

    mask = ((Ab + Sb) > 0).astype(f32)
    inv_deg = 1.0 / (jnp.sum(mask, axis=1, keepdims=True) + 1e-6)

    e0 = emb_ref[0:1, :]
    e1 = emb_ref[1:2, :]

    gi = lax.broadcasted_iota(jnp.int32, (G, G), 0)
    gj = lax.broadcasted_iota(jnp.int32, (G, G), 1)
    same = (gi // V) == (gj // V)
    triu_bd = (same & (gj > gi)).astype(f32)
    tril_bd = (same & (gi > gj)).astype(f32)
    cmats = {
        'right_incl': (same & (gi <= gj)).astype(f32),
        'left_incl': (same & (gj <= gi)).astype(f32),
        'triu_bd': triu_bd,
        'tril_bd': tril_bd,
        'iota_col': lax.broadcasted_iota(jnp.int32, (G, 1), 0).astype(f32),
        'iota_row': lax.broadcasted_iota(jnp.int32, (1, G), 1).astype(f32),
    }

    tU = mask * triu_bd
    tL = mask * tril_bd
    rowU, colU, fpU_col, rowUT, colUT, fpU_row = _onehots(
        tU, tL, cmats, _EU, _WU, f32, transposed=True)
    row_oh = jnp.concatenate([rowU, colU], axis=0)
    col_oh = jnp.concatenate([colU, rowU], axis=0)
    row_ohT = jnp.concatenate([rowUT, colUT], axis=1)
    oh_rc = jnp.concatenate([row_oh, col_oh], axis=1)

    tS = Sb * triu_bd
    tSL = Sb * tril_bd
    rowS, colS, fpS_col, rowST, colST, fpS_row = _onehots(
        tS, tSL, cmats, _ES, _WS, f32, transposed=True)
    ohSW = (jnp.abs(fpS_col - fpU_row) < 0.5).astype(f32)
    ohSW_T = (jnp.abs(fpU_col - fpS_row) < 0.5).astype(f32)

    Sval_u = jnp.dot(ohSW_T, jnp.ones((_ES, 1), f32),
                     preferred_element_type=f32)
    Sval = jnp.concatenate([Sval_u, Sval_u], axis=0)
    s_e = e0 + Sval * (e1 - e0)

    for l in range(_NUM_LAYERS):
        uv = jnp.dot(x_all, wuv_ref[l], preferred_element_type=f32)
        Ux = uv[:, :H]
        Vx = uv[:, H:]
        ge = jnp.dot(s_e, wge_ref[l], preferred_element_type=f32)
        gates = jax.nn.sigmoid(ge[:, :H])
        sE0 = ge[:, H:]
        Vxc = _dot2(col_oh, Vx, f32)
        contrib = gates * Vxc
        agg = _dot2(row_ohT, contrib, f32)
        x_all = jnp.maximum(Ux + agg * inv_deg, 0.0)
        e12 = jnp.dot(x_all, we12_ref[l], preferred_element_type=f32)
        e12s = jnp.concatenate([e12[:, :H], e12[:, H:]], axis=0)
        xE12e = _dot2(oh_rc, e12s, f32)
        s_e = jnp.maximum(sE0 + xE12e, 0.0)

    ks = lax.broadcasted_iota(jnp.int32, (_ES, G), 0)
    gs = lax.broadcasted_iota(jnp.int32, (_ES, G), 1)
    ind_sw = ((gs // V) == (ks // _WS)).astype(f32)
    kb = lax.broadcasted_iota(jnp.int32, (_EB, G), 0)
    gb = lax.broadcasted_iota(jnp.int32, (_EB, G), 1)
    ind_br = ((gb // V) == (kb // _WB)).astype(f32)
    xg_sw = _dot2(ind_sw, x_all, f32)
    xg_br = _dot2(ind_br, x_all, f32)

    sw = jnp.dot(ohSW, s_e[:_EU], preferred_element_type=f32)
    x1 = jnp.dot(rowS, x_all, preferred_element_type=f32)
    x2 = jnp.dot(colS, x_all, preferred_element_type=f32)
    smlp_in = jnp.concatenate([sw, x1, x2, xg_sw], axis=1)
    hs = jnp.maximum(
        jnp.dot(smlp_in, sW1_ref[...], preferred_element_type=f32)
        + sb1_ref[...], 0.0)
    s_out = jnp.dot(hs, sW2_ref[...],
                    preferred_element_type=f32) + sb2_ref[...]

    tA = Ab * triu_bd
    tAL = Ab * tril_bd
    rowA, colA, _ = _onehots(tA, tAL, cmats, _EB, _WB, f32)
    xb = jnp.dot(rowA, x_all, preferred_element_type=f32)
    xe = jnp.dot(colA, x_all, preferred_element_type=f32)
    cmlp_in = jnp.concatenate([xb, xe, xg_br], axis=1)
    hc = jnp.maximum(
        jnp.dot(cmlp_in, cW1_ref[...], preferred_element_type=f32)
        + cb1_ref[...], 0.0)
    c_out = jnp.dot(hc, cW2_ref[...],
                    preferred_element_type=f32) + cb2_ref[...]

    nsw = _NUM_SW
    nbr = _V - 1
    zeros47 = jnp.zeros((nbr, 1), f32)
    for nb in range(_NB):
        co = c_out[nb * _WB:nb * _WB + nbr]
        so = s_out[nb * _WS:nb * _WS + nsw]
        col = jnp.concatenate([
            co[:, 0:1], so[:, 1:2],
            zeros47, jax.nn.sigmoid(so[:, 0:1]),
            co[:, 1:2], so[:, 2:3],
            co[:, 2:3], so[:, 3:4],
        ], axis=0)
        out_ref[0, :, nb:nb + 1] = col


@jax.jit
def kernel(x, A, S, params):
    f32 = jnp.float32
    H = _H
    lp = params['layers']
    wuv = jnp.stack([jnp.concatenate([l['U'], l['Vm']], axis=1) for l in lp])
    wge = jnp.stack([jnp.concatenate([l['G'], l['E0']], axis=1) for l in lp])
    we12 = jnp.stack([jnp.concatenate([l['E1'], l['E2']], axis=1) for l in lp])
    emb = params['embed']
    sW1 = params['smlp_W1']
    sb1 = params['smlp_b1'].reshape(1, 4 * H)
    sW2 = jnp.zeros((4 * H, 8), f32).at[:, :4].set(params['smlp_W2'])
    sb2 = jnp.zeros((1, 8), f32).at[0, :4].set(params['smlp_b2'])
    cW1 = params['cmlp_W1']
    cb1 = params['cmlp_b1'].reshape(1, 3 * H)
    cW2 = jnp.zeros((3 * H, 8), f32).at[:, :3].set(params['cmlp_W2'])
    cb2 = jnp.zeros((1, 8), f32).at[0, :3].set(params['cmlp_b2'])

    nprog = _B // _NB
    A4 = A.reshape(nprog, _NB, _V, _V)
    S4 = S.reshape(nprog, _NB, _V, _V)
    Abig = jnp.zeros((nprog, _G, _G), f32)
    Sbig = jnp.zeros((nprog, _G, _G), f32)
    for nb in range(_NB):
        sl = slice(nb * _V, (nb + 1) * _V)
        Abig = Abig.at[:, sl, sl].set(A4[:, nb])
        Sbig = Sbig.at[:, sl, sl].set(S4[:, nb])

    grid = (nprog,)
    full = lambda shape: pl.BlockSpec(shape, lambda i: (0,) * len(shape))
    out = pl.pallas_call(
        _fwd_kernel,
        grid=grid,
        in_specs=[
            pl.BlockSpec((_NB, _V, _H), lambda i: (i, 0, 0)),
            pl.BlockSpec((1, _G, _G), lambda i: (i, 0, 0)),
            pl.BlockSpec((1, _G, _G), lambda i: (i, 0, 0)),
            full((2, H)),
            full((_NUM_LAYERS, H, 2 * H)),
            full((_NUM_LAYERS, H, 2 * H)),
            full((_NUM_LAYERS, H, 2 * H)),
            full((4 * H, 4 * H)),
            full((1, 4 * H)),
            full((4 * H, 8)),
            full((1, 8)),
            full((3 * H, 3 * H)),
            full((1, 3 * H)),
            full((3 * H, 8)),
            full((1, 8)),
        ],
        out_specs=pl.BlockSpec((1, 4 * _M_EDGES, _NB), lambda i: (i, 0, 0)),
        out_shape=jax.ShapeDtypeStruct((nprog, 4 * _M_EDGES, _NB), f32),
        compiler_params=pltpu.CompilerParams(
            dimension_semantics=("parallel",)),
    )(x, Abig, Sbig, emb, wuv, wge, we12, sW1, sb1, sW2, sb2,
      cW1, cb1, cW2, cb2)
    return out.transpose(0, 2, 1).reshape(_B, 4 * _M_EDGES)
